# baseline (speedup 1.0000x reference)
import jax
import jax.numpy as jnp
from jax.experimental import pallas as pl
from jax.experimental.pallas import tpu as pltpu

_B = 1024
_C = 100
_BETA = 0.7
_LAMBDA1 = 3.0


def _elr_loss_kernel(idx_ref, out_ref, lab_ref, tgt_hbm, loss_ref, old_vmem, sem):
    row0 = idx_ref[0] * _B
    copy = pltpu.make_async_copy(
        tgt_hbm.at[pl.ds(row0, _B), :], old_vmem, sem
    )
    copy.start()
    o = out_ref[...]
    lab = lab_ref[...]
    m = jnp.max(o, axis=1, keepdims=True)
    e = jnp.exp(o - m)
    s = jnp.sum(e, axis=1, keepdims=True)
    y_pred = jnp.clip(e / s, 0.0001, 1.0 - 0.0001)
    y_norm = y_pred / jnp.sum(y_pred, axis=1, keepdims=True)
    logp = (o - m) - jnp.log(s)
    cols = jax.lax.broadcasted_iota(jnp.int32, (_B, _C), 1)
    picked = jnp.where(cols == lab, logp, 0.0)
    ce = -jnp.sum(picked) / _B
    copy.wait()
    old = old_vmem[...]
    new = _BETA * old + (1.0 - _BETA) * y_norm
    elr = jnp.sum(jnp.log(1.0 - jnp.sum(new * y_pred, axis=1))) / _B
    loss_ref[0, 0] = ce + _LAMBDA1 * elr


def kernel(index, output, label, target):
    idx = jnp.asarray(index, dtype=jnp.int32).reshape((1,))
    lab2d = label.astype(jnp.int32).reshape(_B, 1)
    grid_spec = pltpu.PrefetchScalarGridSpec(
        num_scalar_prefetch=1,
        grid=(1,),
        in_specs=[
            pl.BlockSpec((_B, _C), lambda i, idx_ref: (0, 0)),
            pl.BlockSpec((_B, 1), lambda i, idx_ref: (0, 0)),
            pl.BlockSpec(memory_space=pl.ANY),
        ],
        out_specs=pl.BlockSpec(
            (1, 1), lambda i, idx_ref: (0, 0), memory_space=pltpu.SMEM
        ),
        scratch_shapes=[
            pltpu.VMEM((_B, _C), jnp.float32),
            pltpu.SemaphoreType.DMA,
        ],
    )
    loss = pl.pallas_call(
        _elr_loss_kernel,
        grid_spec=grid_spec,
        out_shape=jax.ShapeDtypeStruct((1, 1), jnp.float32),
    )(idx, output, lab2d, target)
    return loss[0, 0]


# speedup vs baseline: 113.6217x; 113.6217x over previous
import jax
import jax.numpy as jnp
from jax.experimental import pallas as pl
from jax.experimental.pallas import tpu as pltpu

_B = 1024
_C = 100
_BETA = 0.7
_LAMBDA1 = 3.0


def _elr_loss_kernel(idx_ref, out_ref, lab_ref, tgt_hbm, loss_ref, old_vmem, sem):
    col0 = idx_ref[0] * _B
    copy = pltpu.make_async_copy(
        tgt_hbm.at[:, pl.ds(col0, _B)], old_vmem, sem
    )
    copy.start()
    o = out_ref[...]
    lab = lab_ref[...]
    m = jnp.max(o, axis=0, keepdims=True)
    e = jnp.exp(o - m)
    s = jnp.sum(e, axis=0, keepdims=True)
    y_pred = jnp.clip(e / s, 0.0001, 1.0 - 0.0001)
    y_norm = y_pred / jnp.sum(y_pred, axis=0, keepdims=True)
    logp = (o - m) - jnp.log(s)
    rows = jax.lax.broadcasted_iota(jnp.int32, (_C, _B), 0)
    picked = jnp.where(rows == lab, logp, 0.0)
    ce = -jnp.sum(picked) / _B
    copy.wait()
    old = old_vmem[...]
    new = _BETA * old + (1.0 - _BETA) * y_norm
    elr = jnp.sum(jnp.log(1.0 - jnp.sum(new * y_pred, axis=0))) / _B
    loss_ref[0, 0] = ce + _LAMBDA1 * elr


def kernel(index, output, label, target):
    idx = jnp.asarray(index, dtype=jnp.int32).reshape((1,))
    ot = output.T
    tt = target.T
    lab2d = label.astype(jnp.int32).reshape(1, _B)
    grid_spec = pltpu.PrefetchScalarGridSpec(
        num_scalar_prefetch=1,
        grid=(1,),
        in_specs=[
            pl.BlockSpec((_C, _B), lambda i, idx_ref: (0, 0)),
            pl.BlockSpec((1, _B), lambda i, idx_ref: (0, 0)),
            pl.BlockSpec(memory_space=pl.ANY),
        ],
        out_specs=pl.BlockSpec(
            (1, 1), lambda i, idx_ref: (0, 0), memory_space=pltpu.SMEM
        ),
        scratch_shapes=[
            pltpu.VMEM((_C, _B), jnp.float32),
            pltpu.SemaphoreType.DMA,
        ],
    )
    loss = pl.pallas_call(
        _elr_loss_kernel,
        grid_spec=grid_spec,
        out_shape=jax.ShapeDtypeStruct((1, 1), jnp.float32),
    )(idx, ot, lab2d, tt)
    return loss[0, 0]
